# baseline (speedup 1.0000x reference)
import functools

import jax
import jax.numpy as jnp
from jax import lax
from jax.experimental import pallas as pl
from jax.experimental.pallas import tpu as pltpu
from jax.experimental.pallas import tpu_sc as plsc

B = 2048
K = 1024
D = 64
BLK = 256

_SC_INFO = plsc.get_sparse_core_info()
_NC = _SC_INFO.num_cores
_NS = _SC_INFO.num_subcores
_NW = _NC * _NS
_BPW = B // _NW


def _split_bf16(x):
    hi = x.astype(jnp.bfloat16)
    lo = (x - hi.astype(jnp.float32)).astype(jnp.bfloat16)
    return hi, lo


def _mm_t(a, b):
    return jax.lax.dot_general(
        a, b, (((1,), (1,)), ((), ())),
        preferred_element_type=jnp.float32)


def _argmin_block(ze_blk, embt_ref, idx_ref, ehi_ref, elo_ref, esq_ref):
    @pl.when(pl.program_id(0) == 0)
    def _prep():
        embt = embt_ref[...]
        ehi, elo = _split_bf16(embt)
        ehi_ref[...] = ehi
        elo_ref[...] = elo
        esq_ref[...] = jnp.sum(embt * embt, axis=1, keepdims=True)

    ze = ze_blk[...]
    ze_hi, ze_lo = _split_bf16(ze)
    dots = (_mm_t(ehi_ref[...], ze_hi)
            + (_mm_t(elo_ref[...], ze_hi) + _mm_t(ehi_ref[...], ze_lo)))
    scores = esq_ref[...] - 2.0 * dots
    mins = jnp.min(scores, axis=0, keepdims=True)
    row = jax.lax.broadcasted_iota(jnp.int32, scores.shape, 0)
    idx = jnp.min(jnp.where(scores == mins, row, D), axis=0, keepdims=True)
    idx_ref[...] = idx.reshape(1, 1, BLK)


def _tc_argmin(ze, embt):
    grid = (B // BLK,)
    return pl.pallas_call(
        _argmin_block,
        grid=grid,
        in_specs=[
            pl.BlockSpec((BLK, K), lambda i: (i, 0)),
            pl.BlockSpec((D, K), lambda i: (0, 0)),
        ],
        out_specs=pl.BlockSpec((1, 1, BLK), lambda i: (i, 0, 0)),
        out_shape=jax.ShapeDtypeStruct((B // BLK, 1, BLK), jnp.int32),
        scratch_shapes=[
            pltpu.VMEM((D, K), jnp.bfloat16),
            pltpu.VMEM((D, K), jnp.bfloat16),
            pltpu.VMEM((D, 1), jnp.float32),
        ],
        compiler_params=pltpu.CompilerParams(
            dimension_semantics=("arbitrary",)),
    )(ze, embt)


_NCH = 4
_CH = _BPW // _NCH


@functools.partial(
    pl.kernel,
    mesh=plsc.VectorSubcoreMesh(core_axis_name="c", subcore_axis_name="s"),
    out_type=jax.ShapeDtypeStruct((B, K), jnp.float32),
    scratch_types=[
        pltpu.VMEM((_BPW,), jnp.int32),
        pltpu.VMEM((_BPW, K), jnp.float32),
        [pltpu.SemaphoreType.DMA] * _NCH,
        [pltpu.SemaphoreType.DMA] * _NCH,
    ],
)
def _sc_gather(ze_hbm, idx_hbm, out_hbm, idx_v, rows_v, gsems, wsems):
    wid = lax.axis_index("s") * _NC + lax.axis_index("c")
    base = wid * _BPW
    pltpu.sync_copy(idx_hbm.at[pl.ds(base, _BPW)], idx_v)
    handles = []
    for i in range(_NCH):
        handles.append(pltpu.async_copy(
            ze_hbm.at[idx_v.at[pl.ds(i * _CH, _CH)]],
            rows_v.at[pl.ds(i * _CH, _CH)], gsems[i]))
    writes = []
    for i in range(_NCH):
        handles[i].wait()
        writes.append(pltpu.async_copy(
            rows_v.at[pl.ds(i * _CH, _CH)],
            out_hbm.at[pl.ds(base + i * _CH, _CH)], wsems[i]))
    for w in writes:
        w.wait()


@functools.partial(jax.jit, static_argnames=())
def kernel(ze, emb):
    idx = _tc_argmin(ze, emb.T).reshape(B)
    return _sc_gather(ze, idx)


# speedup vs baseline: 3.1302x; 3.1302x over previous
import functools

import jax
import jax.numpy as jnp
from jax.experimental import pallas as pl
from jax.experimental.pallas import tpu as pltpu

B = 2048
K = 1024
D = 64
BLK = 256


def _split_bf16(x):
    hi = x.astype(jnp.bfloat16)
    lo = (x - hi.astype(jnp.float32)).astype(jnp.bfloat16)
    return hi, lo


def _mm(a, b):
    return jax.lax.dot_general(
        a, b, (((1,), (0,)), ((), ())),
        preferred_element_type=jnp.float32)


def _vq_block(ze_blk, emb_ref, ze_head_ref, out_ref, ehi_ref, elo_ref,
              esq_ref):
    @pl.when(pl.program_id(0) == 0)
    def _prep():
        emb = emb_ref[...]
        ehi, elo = _split_bf16(emb)
        ehi_ref[...] = ehi
        elo_ref[...] = elo
        esq_ref[...] = jnp.sum(emb * emb, axis=0, keepdims=True)

    ze = ze_blk[...]
    ze_hi, ze_lo = _split_bf16(ze)
    dots = (_mm(ze_hi, ehi_ref[...])
            + (_mm(ze_hi, elo_ref[...]) + _mm(ze_lo, ehi_ref[...])))
    scores = esq_ref[...] - 2.0 * dots
    mins = jnp.min(scores, axis=1, keepdims=True)
    col = jax.lax.broadcasted_iota(jnp.int32, scores.shape, 1)
    idx = jnp.min(jnp.where(scores == mins, col, D), axis=1, keepdims=True)
    onehot = (col == idx).astype(jnp.float32)
    out_ref[...] = _mm(onehot, ze_head_ref[...])


@functools.partial(jax.jit, static_argnames=())
def kernel(ze, emb):
    grid = (B // BLK,)
    return pl.pallas_call(
        _vq_block,
        grid=grid,
        in_specs=[
            pl.BlockSpec((BLK, K), lambda i: (i, 0)),
            pl.BlockSpec((K, D), lambda i: (0, 0)),
            pl.BlockSpec((D, K), lambda i: (0, 0)),
        ],
        out_specs=pl.BlockSpec((BLK, K), lambda i: (i, 0)),
        out_shape=jax.ShapeDtypeStruct((B, K), jnp.float32),
        scratch_shapes=[
            pltpu.VMEM((K, D), jnp.bfloat16),
            pltpu.VMEM((K, D), jnp.bfloat16),
            pltpu.VMEM((1, D), jnp.float32),
        ],
        compiler_params=pltpu.CompilerParams(
            dimension_semantics=("arbitrary",)),
    )(ze, emb, ze)
